# baseline (speedup 1.0000x reference)
import jax
import jax.numpy as jnp
from jax.experimental import pallas as pl
from jax.experimental.pallas import tpu as pltpu
from jax.experimental.pallas import tpu_sc as plsc

M, B, D = 65536, 8192, 256
T = M + B

BLK = 4096
NM = M // BLK
NB = B // BLK


def _tc_body(k_ref, nk_ref, ok_ref):
    i = pl.program_id(0)

    @pl.when(i < NM)
    def _():
        ok_ref[...] = k_ref[...]

    @pl.when(i >= NM)
    def _():
        ok_ref[...] = nk_ref[...]


def _tc_concat(keys, new_keys):
    return pl.pallas_call(
        _tc_body,
        grid=(NM + NB,),
        in_specs=[
            pl.BlockSpec((BLK, D), lambda i: (jnp.minimum(i, NM - 1), 0)),
            pl.BlockSpec((BLK, D), lambda i: (jnp.maximum(i - NM, 0), 0)),
        ],
        out_specs=pl.BlockSpec((BLK, D), lambda i: (i, 0)),
        out_shape=jax.ShapeDtypeStruct((T, D), keys.dtype),
    )(keys, new_keys)


CH = 2048
HALF_OLD = M // 2
HALF_NEW = B // 2
N_OLD = HALF_OLD // CH
N_NEW = HALF_NEW // CH


def _sc_concat(vals, new_vals):
    mesh = plsc.ScalarSubcoreMesh(axis_name="core", num_cores=2)

    @pl.kernel(
        out_type=jax.ShapeDtypeStruct((T, D), vals.dtype),
        mesh=mesh,
        scratch_types=[
            pltpu.VMEM_SHARED((CH, D), vals.dtype),
            pltpu.VMEM_SHARED((CH, D), vals.dtype),
            pltpu.SemaphoreType.DMA((2,)),
            pltpu.SemaphoreType.DMA((2,)),
        ],
    )
    def sc_copy(v_hbm, nv_hbm, ov_hbm, buf0, buf1, in_sems, out_sems):
        core = jax.lax.axis_index("core")
        old_base = core * HALF_OLD
        new_base = core * HALF_NEW
        bufs = (buf0, buf1)

        chunks = []
        for c in range(N_OLD):
            off = old_base + c * CH
            chunks.append((v_hbm, off, ov_hbm, off))
        for c in range(N_NEW):
            off = new_base + c * CH
            chunks.append((nv_hbm, off, ov_hbm, M + off))
        n = len(chunks)

        in_cp = [None] * n
        out_cp = [None] * n
        for i, (src, soff, dst, doff) in enumerate(chunks):
            b = i % 2
            if i >= 2:
                out_cp[i - 2].wait()
            in_cp[i] = pltpu.make_async_copy(
                src.at[pl.ds(soff, CH), :], bufs[b], in_sems.at[b])
            in_cp[i].start()
            if i >= 1:
                pb = (i - 1) % 2
                in_cp[i - 1].wait()
                out_cp[i - 1] = pltpu.make_async_copy(
                    bufs[pb], chunks[i - 1][2].at[pl.ds(chunks[i - 1][3], CH), :],
                    out_sems.at[pb])
                out_cp[i - 1].start()
        in_cp[n - 1].wait()
        out_cp[n - 1] = pltpu.make_async_copy(
            bufs[(n - 1) % 2],
            chunks[n - 1][2].at[pl.ds(chunks[n - 1][3], CH), :],
            out_sems.at[(n - 1) % 2])
        out_cp[n - 1].start()
        out_cp[n - 2].wait()
        out_cp[n - 1].wait()

    return sc_copy(vals, new_vals)


def kernel(keys, vals, new_keys, new_vals):
    out_keys = _tc_concat(keys, new_keys)
    out_vals = _sc_concat(vals, new_vals)
    return (out_keys, out_vals)


# speedup vs baseline: 1.0222x; 1.0222x over previous
import jax
import jax.numpy as jnp
from jax.experimental import pallas as pl
from jax.experimental.pallas import tpu as pltpu
from jax.experimental.pallas import tpu_sc as plsc

M, B, D = 65536, 8192, 256
T = M + B

BLK = 4096
NM = M // BLK
NB = B // BLK


def _tc_body(k_ref, nk_ref, ok_ref):
    i = pl.program_id(0)

    @pl.when(i < NM)
    def _():
        ok_ref[...] = k_ref[...]

    @pl.when(i >= NM)
    def _():
        ok_ref[...] = nk_ref[...]


def _tc_concat(keys, new_keys):
    return pl.pallas_call(
        _tc_body,
        grid=(NM + NB,),
        in_specs=[
            pl.BlockSpec((BLK, D), lambda i: (jnp.minimum(i, NM - 1), 0)),
            pl.BlockSpec((BLK, D), lambda i: (jnp.maximum(i - NM, 0), 0)),
        ],
        out_specs=pl.BlockSpec((BLK, D), lambda i: (i, 0)),
        out_shape=jax.ShapeDtypeStruct((T, D), keys.dtype),
    )(keys, new_keys)


CH = 2048
NBUF = 3
HALF_OLD = M // 2
HALF_NEW = B // 2
N_OLD = HALF_OLD // CH
N_NEW = HALF_NEW // CH


def _sc_concat(vals, new_vals):
    mesh = plsc.ScalarSubcoreMesh(axis_name="core", num_cores=2)

    @pl.kernel(
        out_type=jax.ShapeDtypeStruct((T, D), vals.dtype),
        mesh=mesh,
        scratch_types=[
            [pltpu.VMEM_SHARED((CH, D), vals.dtype) for _ in range(NBUF)],
            pltpu.SemaphoreType.DMA((NBUF,)),
            pltpu.SemaphoreType.DMA((NBUF,)),
        ],
    )
    def sc_copy(v_hbm, nv_hbm, ov_hbm, bufs, in_sems, out_sems):
        core = jax.lax.axis_index("core")
        old_base = core * HALF_OLD
        new_base = core * HALF_NEW

        chunks = []
        for c in range(N_OLD):
            off = old_base + c * CH
            chunks.append((v_hbm, off, off))
        for c in range(N_NEW):
            off = new_base + c * CH
            chunks.append((nv_hbm, off, M + off))
        n = len(chunks)

        in_cp = [None] * n
        out_cp = [None] * n

        def start_in(i):
            src, soff, _ = chunks[i]
            b = i % NBUF
            in_cp[i] = pltpu.make_async_copy(
                src.at[pl.ds(soff, CH), :], bufs[b], in_sems.at[b])
            in_cp[i].start()

        def start_out(i):
            _, _, doff = chunks[i]
            b = i % NBUF
            out_cp[i] = pltpu.make_async_copy(
                bufs[b], ov_hbm.at[pl.ds(doff, CH), :], out_sems.at[b])
            out_cp[i].start()

        for i in range(min(NBUF, n)):
            start_in(i)
        for i in range(n):
            in_cp[i].wait()
            start_out(i)
            j = i + NBUF - 1
            if i >= 1 and j < n:
                out_cp[i - 1].wait()
                start_in(j)
        for i in range(max(0, n - NBUF + 1), n):
            out_cp[i].wait()

    return sc_copy(vals, new_vals)


def kernel(keys, vals, new_keys, new_vals):
    out_keys = _tc_concat(keys, new_keys)
    out_vals = _sc_concat(vals, new_vals)
    return (out_keys, out_vals)
